# baseline (speedup 1.0000x reference)
import jax
import jax.numpy as jnp
from jax import lax
from jax.experimental import pallas as pl
from jax.experimental.pallas import tpu as pltpu
from jax.experimental.pallas import tpu_sc as plsc

N = 10000
E = 320000
D = 128
H = 128

NC = 2
NS = 16
NW = NC * NS
CHUNK = 128
CH = 80
HCH = CH // 2
EPW = CH * CHUNK
E_PAD = NW * EPW
N_PAD = 10240
DA = 144
RPS = N_PAD // NS


def _prologue_body(x_ref, g_ref, b_ref, wl_ref, wr_ref, bl_ref,
                   ycat_ref, r_ref):
    scale = g_ref[...] * (1.0 / jnp.sqrt(1.0 + 1e-5))
    xt = jnp.maximum(x_ref[...], 0.0) * scale[None, :] + b_ref[...][None, :]
    ycat_ref[:, :D] = jnp.dot(xt, wl_ref[...],
                              preferred_element_type=jnp.float32)
    col = lax.broadcasted_iota(jnp.int32, (x_ref.shape[0], DA - D), 1)
    ycat_ref[:, D:] = jnp.where(col == 0, 1.0, 0.0)
    r_ref[...] = (jnp.dot(xt, wr_ref[...], preferred_element_type=jnp.float32)
                  + bl_ref[...][None, :])


def _sc_body(y_hbm, src2_hbm, dst2_hbm, agg_hbm,
             sidx_v, didx_v, rows_v, zbuf_v, agg_sh):
    cid = lax.axis_index("c")
    sid = lax.axis_index("s")
    wid = sid * NC + cid

    zero16 = jnp.zeros((16,), jnp.float32)
    for i in range(32):
        for j in range(DA // 16):
            zbuf_v[i, pl.ds(j * 16, 16)] = zero16

    def zloop(k, c):
        pltpu.sync_copy(zbuf_v, agg_sh.at[pl.ds(sid * RPS + k * 32, 32)])
        return c
    lax.fori_loop(0, RPS // 32, zloop, 0)
    plsc.subcore_barrier()

    def half(h, c):
        cbase = wid * CH + h * HCH
        pltpu.sync_copy(src2_hbm.at[pl.ds(cbase, HCH)], sidx_v)
        pltpu.sync_copy(dst2_hbm.at[pl.ds(cbase, HCH)], didx_v)

        def eloop(i, c2):
            pltpu.sync_copy(y_hbm.at[sidx_v.at[i]], rows_v)
            pltpu.sync_copy(rows_v, agg_sh.at[didx_v.at[i]], add=True)
            return c2
        lax.fori_loop(0, HCH, eloop, 0)
        return c
    lax.fori_loop(0, 2, half, 0)
    plsc.subcore_barrier()

    pltpu.sync_copy(agg_sh.at[pl.ds(sid * RPS, RPS)],
                    agg_hbm.at[cid].at[pl.ds(sid * RPS, RPS)])


def _epilogue_body(a0_ref, a1_ref, r_ref, out_ref):
    s = a0_ref[...] + a1_ref[...]
    cnt = jnp.maximum(s[:, D:D + 1], 1.0)
    out = s[:, :D] / cnt + r_ref[...]
    nrm = jnp.sqrt(jnp.sum(out * out, axis=1, keepdims=True))
    out_ref[...] = out / jnp.maximum(nrm, 1e-12)


def kernel(x, edge_index, W1l, b1l, W1r, bn_gamma, bn_beta, W2l, b2l, W2r):
    del W1l, b1l, W1r

    src = edge_index[0]
    dst = edge_index[1]
    pad = E_PAD - E
    srcp = jnp.concatenate([src, jnp.zeros((pad,), jnp.int32)])
    dstp = jnp.concatenate([dst, jnp.full((pad,), N_PAD - 1, jnp.int32)])
    src2 = srcp.reshape(NW * CH, CHUNK)
    dst2 = dstp.reshape(NW * CH, CHUNK)

    ycat, r = pl.pallas_call(
        _prologue_body,
        out_shape=(
            jax.ShapeDtypeStruct((N, DA), jnp.float32),
            jax.ShapeDtypeStruct((N, H), jnp.float32),
        ),
    )(x, bn_gamma, bn_beta, W2l, W2r, b2l)

    mesh = plsc.VectorSubcoreMesh(core_axis_name="c", subcore_axis_name="s")
    agg2 = pl.kernel(
        _sc_body,
        out_type=jax.ShapeDtypeStruct((NC, N_PAD, DA), jnp.float32),
        mesh=mesh,
        scratch_types=[
            pltpu.VMEM((HCH, CHUNK), jnp.int32),
            pltpu.VMEM((HCH, CHUNK), jnp.int32),
            pltpu.VMEM((CHUNK, DA), jnp.float32),
            pltpu.VMEM((32, DA), jnp.float32),
            pltpu.VMEM_SHARED((N_PAD, DA), jnp.float32),
        ],
        compiler_params=pltpu.CompilerParams(use_tc_tiling_on_sc=False),
    )(ycat, src2, dst2)

    out = pl.pallas_call(
        _epilogue_body,
        out_shape=jax.ShapeDtypeStruct((N, H), jnp.float32),
    )(agg2[0, :N], agg2[1, :N], r)
    return out


# speedup vs baseline: 1.7779x; 1.7779x over previous
import jax
import jax.numpy as jnp
from jax import lax
from jax.experimental import pallas as pl
from jax.experimental.pallas import tpu as pltpu
from jax.experimental.pallas import tpu_sc as plsc

N = 10000
E = 320000
D = 128
H = 128

NC = 2
NS = 16
NW = NC * NS
CHUNK = 128
CH = 80
EPW = CH * CHUNK
E_PAD = NW * EPW
N_PAD = 10240
DA = 144
W = DA // 2
CC = D - W
RPS = N_PAD // NS
SRS = N // NS


def _prologue_body(x_ref, g_ref, b_ref, wl_ref, wr_ref, bl_ref,
                   ycat_ref, r_ref):
    scale = g_ref[...] * (1.0 / jnp.sqrt(1.0 + 1e-5))
    xt = jnp.maximum(x_ref[...], 0.0) * scale[None, :] + b_ref[...][None, :]
    ycat_ref[:, :D] = jnp.dot(xt, wl_ref[...],
                              preferred_element_type=jnp.float32)
    col = lax.broadcasted_iota(jnp.int32, (x_ref.shape[0], DA - D), 1)
    ycat_ref[:, D:] = jnp.where(col == 0, 1.0, 0.0)
    r_ref[...] = (jnp.dot(xt, wr_ref[...], preferred_element_type=jnp.float32)
                  + bl_ref[...][None, :])


def _sc_body(y_hbm, src2_hbm, dst2_hbm, agg_hbm,
             sidx_v, didx_v, rows_v, zbuf_v, y_sh, agg_sh):
    cid = lax.axis_index("c")
    sid = lax.axis_index("s")
    wid = sid * NC + cid

    zero16 = jnp.zeros((16,), jnp.float32)
    for i in range(32):
        for j in range(W // 8 // 2):
            zbuf_v[i, pl.ds(j * 16, 16)] = zero16
        zbuf_v[i, pl.ds(W - 16, 16)] = zero16

    pltpu.sync_copy(src2_hbm.at[pl.ds(wid * CH, CH)], sidx_v)
    pltpu.sync_copy(dst2_hbm.at[pl.ds(wid * CH, CH)], didx_v)

    for p in range(2):
        pltpu.sync_copy(
            y_hbm.at[pl.ds(sid * SRS, SRS), pl.ds(p * W, W)],
            y_sh.at[pl.ds(sid * SRS, SRS)])

        def zloop(k, c):
            pltpu.sync_copy(zbuf_v,
                            agg_sh.at[pl.ds(sid * RPS + k * 32, 32)])
            return c
        lax.fori_loop(0, RPS // 32, zloop, 0)
        plsc.subcore_barrier()

        def eloop(i, c2):
            pltpu.sync_copy(y_sh.at[sidx_v.at[i]], rows_v)
            pltpu.sync_copy(rows_v, agg_sh.at[didx_v.at[i]], add=True)
            return c2
        lax.fori_loop(0, CH, eloop, 0)
        plsc.subcore_barrier()

        pltpu.sync_copy(agg_sh.at[pl.ds(sid * RPS, RPS)],
                        agg_hbm.at[cid].at[p].at[pl.ds(sid * RPS, RPS)])


def _epilogue_body(a00_ref, a01_ref, a10_ref, a11_ref, r_ref, out_ref):
    sa = a00_ref[...] + a10_ref[...]
    sb = a01_ref[...] + a11_ref[...]
    cnt = jnp.maximum(sb[:, CC:CC + 1], 1.0)
    mean = jnp.concatenate([sa, sb[:, :CC]], axis=1) / cnt
    out = mean + r_ref[...]
    nrm = jnp.sqrt(jnp.sum(out * out, axis=1, keepdims=True))
    out_ref[...] = out / jnp.maximum(nrm, 1e-12)


def kernel(x, edge_index, W1l, b1l, W1r, bn_gamma, bn_beta, W2l, b2l, W2r):
    del W1l, b1l, W1r

    src = edge_index[0]
    dst = edge_index[1]
    pad = E_PAD - E
    srcp = jnp.concatenate([src, jnp.zeros((pad,), jnp.int32)])
    dstp = jnp.concatenate([dst, jnp.full((pad,), N_PAD - 1, jnp.int32)])
    src2 = srcp.reshape(NW * CH, CHUNK)
    dst2 = dstp.reshape(NW * CH, CHUNK)

    ycat, r = pl.pallas_call(
        _prologue_body,
        out_shape=(
            jax.ShapeDtypeStruct((N, DA), jnp.float32),
            jax.ShapeDtypeStruct((N, H), jnp.float32),
        ),
    )(x, bn_gamma, bn_beta, W2l, W2r, b2l)

    mesh = plsc.VectorSubcoreMesh(core_axis_name="c", subcore_axis_name="s")
    agg2 = pl.kernel(
        _sc_body,
        out_type=jax.ShapeDtypeStruct((NC, 2, N_PAD, W), jnp.float32),
        mesh=mesh,
        scratch_types=[
            pltpu.VMEM((CH, CHUNK), jnp.int32),
            pltpu.VMEM((CH, CHUNK), jnp.int32),
            pltpu.VMEM((CHUNK, W), jnp.float32),
            pltpu.VMEM((32, W), jnp.float32),
            pltpu.VMEM_SHARED((N_PAD, W), jnp.float32),
            pltpu.VMEM_SHARED((N_PAD, W), jnp.float32),
        ],
        compiler_params=pltpu.CompilerParams(use_tc_tiling_on_sc=False),
    )(ycat, src2, dst2)

    out = pl.pallas_call(
        _epilogue_body,
        out_shape=jax.ShapeDtypeStruct((N, H), jnp.float32),
    )(agg2[0, 0, :N], agg2[0, 1, :N], agg2[1, 0, :N], agg2[1, 1, :N], r)
    return out


# speedup vs baseline: 1.8449x; 1.0377x over previous
import jax
import jax.numpy as jnp
from jax import lax
from jax.experimental import pallas as pl
from jax.experimental.pallas import tpu as pltpu
from jax.experimental.pallas import tpu_sc as plsc

N = 10000
E = 320000
D = 128
H = 128

NC = 2
NS = 16
NW = NC * NS
CHUNK = 128
CH = 80
EPW = CH * CHUNK
E_PAD = NW * EPW
N_PAD = 10240
DA = 144
W = DA // 2
CC = D - W
RPS = N_PAD // NS
SRS = N // NS


def _prologue_body(x_ref, g_ref, b_ref, wl_ref, wr_ref, bl_ref,
                   ycat_ref, r_ref):
    scale = g_ref[...] * (1.0 / jnp.sqrt(1.0 + 1e-5))
    xt = jnp.maximum(x_ref[...], 0.0) * scale[None, :] + b_ref[...][None, :]
    ycat_ref[:, :D] = jnp.dot(xt, wl_ref[...],
                              preferred_element_type=jnp.float32)
    col = lax.broadcasted_iota(jnp.int32, (x_ref.shape[0], DA - D), 1)
    ycat_ref[:, D:] = jnp.where(col == 0, 1.0, 0.0)
    r_ref[...] = (jnp.dot(xt, wr_ref[...], preferred_element_type=jnp.float32)
                  + bl_ref[...][None, :])


def _sc_body(y_hbm, src2_hbm, dst2_hbm, agg_hbm,
             sidx_v, didx_v, rows_v, zbuf_v, y_sh, agg_sh):
    cid = lax.axis_index("c")
    sid = lax.axis_index("s")
    wid = sid * NC + cid

    zero16 = jnp.zeros((16,), jnp.float32)
    for i in range(32):
        for j in range(W // 8 // 2):
            zbuf_v[i, pl.ds(j * 16, 16)] = zero16
        zbuf_v[i, pl.ds(W - 16, 16)] = zero16

    pltpu.sync_copy(src2_hbm.at[pl.ds(wid * CH, CH)], sidx_v)
    pltpu.sync_copy(dst2_hbm.at[pl.ds(wid * CH, CH)], didx_v)

    for p in range(2):
        pltpu.sync_copy(
            y_hbm.at[pl.ds(sid * SRS, SRS), pl.ds(p * W, W)],
            y_sh.at[pl.ds(sid * SRS, SRS)])

        def zloop(k, c):
            pltpu.sync_copy(zbuf_v,
                            agg_sh.at[pl.ds(sid * RPS + k * 32, 32)])
            return c
        lax.fori_loop(0, RPS // 32, zloop, 0)
        plsc.subcore_barrier()

        def eloop(i, c2):
            pltpu.sync_copy(y_sh.at[sidx_v.at[i]], rows_v)
            pltpu.sync_copy(rows_v, agg_sh.at[didx_v.at[i]], add=True)
            return c2
        lax.fori_loop(0, CH, eloop, 0)
        plsc.subcore_barrier()

        pltpu.sync_copy(agg_sh.at[pl.ds(sid * RPS, RPS)],
                        agg_hbm.at[cid].at[p].at[pl.ds(sid * RPS, RPS)])


def _epilogue_body(a_ref, r_ref, out_ref):
    sa = a_ref[0, 0] + a_ref[1, 0]
    sb = a_ref[0, 1] + a_ref[1, 1]
    cnt = jnp.maximum(sb[:, CC:CC + 1], 1.0)
    mean = jnp.concatenate([sa, sb[:, :CC]], axis=1) / cnt
    out = mean + r_ref[...]
    nrm = jnp.sqrt(jnp.sum(out * out, axis=1, keepdims=True))
    out_ref[...] = out / jnp.maximum(nrm, 1e-12)


def kernel(x, edge_index, W1l, b1l, W1r, bn_gamma, bn_beta, W2l, b2l, W2r):
    del W1l, b1l, W1r

    src = edge_index[0]
    dst = edge_index[1]
    pad = E_PAD - E
    srcp = jnp.concatenate([src, jnp.zeros((pad,), jnp.int32)])
    dstp = jnp.concatenate([dst, jnp.full((pad,), N_PAD - 1, jnp.int32)])
    src2 = srcp.reshape(NW * CH, CHUNK)
    dst2 = dstp.reshape(NW * CH, CHUNK)

    ycat, r = pl.pallas_call(
        _prologue_body,
        out_shape=(
            jax.ShapeDtypeStruct((N, DA), jnp.float32),
            jax.ShapeDtypeStruct((N, H), jnp.float32),
        ),
    )(x, bn_gamma, bn_beta, W2l, W2r, b2l)

    mesh = plsc.VectorSubcoreMesh(core_axis_name="c", subcore_axis_name="s")
    agg2 = pl.kernel(
        _sc_body,
        out_type=jax.ShapeDtypeStruct((NC, 2, N_PAD, W), jnp.float32),
        mesh=mesh,
        scratch_types=[
            pltpu.VMEM((CH, CHUNK), jnp.int32),
            pltpu.VMEM((CH, CHUNK), jnp.int32),
            pltpu.VMEM((CHUNK, W), jnp.float32),
            pltpu.VMEM((32, W), jnp.float32),
            pltpu.VMEM_SHARED((N_PAD, W), jnp.float32),
            pltpu.VMEM_SHARED((N_PAD, W), jnp.float32),
        ],
        compiler_params=pltpu.CompilerParams(use_tc_tiling_on_sc=False),
    )(ycat, src2, dst2)

    out = pl.pallas_call(
        _epilogue_body,
        grid=(1,),
        in_specs=[
            pl.BlockSpec((NC, 2, N, W), lambda i: (0, 0, 0, 0)),
            pl.BlockSpec((N, H), lambda i: (0, 0)),
        ],
        out_specs=pl.BlockSpec((N, H), lambda i: (0, 0)),
        out_shape=jax.ShapeDtypeStruct((N, H), jnp.float32),
    )(agg2, r)
    return out


# speedup vs baseline: 2.2486x; 1.2188x over previous
import jax
import jax.numpy as jnp
from jax import lax
from jax.experimental import pallas as pl
from jax.experimental.pallas import tpu as pltpu
from jax.experimental.pallas import tpu_sc as plsc

N = 10000
E = 320000
D = 128
H = 128

NC = 2
NS = 16
NW = NC * NS
CHUNK = 128
CH = 80
HCH = CH // 2
EPW = CH * CHUNK
E_PAD = NW * EPW
N_PAD = 10240
DA = 144
W = DA // 2
CC = D - W
RPS = N_PAD // NS
SRS = N // NS


def _prologue_body(x_ref, g_ref, b_ref, wl_ref, wr_ref, bl_ref,
                   ycat_ref, r_ref):
    scale = g_ref[...] * (1.0 / jnp.sqrt(1.0 + 1e-5))
    xt = jnp.maximum(x_ref[...], 0.0) * scale[None, :] + b_ref[...][None, :]
    ycat_ref[:, :D] = jnp.dot(xt, wl_ref[...],
                              preferred_element_type=jnp.float32)
    col = lax.broadcasted_iota(jnp.int32, (x_ref.shape[0], DA - D), 1)
    ycat_ref[:, D:] = jnp.where(col == 0, 1.0, 0.0)
    r_ref[...] = (jnp.dot(xt, wr_ref[...], preferred_element_type=jnp.float32)
                  + bl_ref[...][None, :])


def _sc_body(y_hbm, src2_hbm, dst2_hbm, agg_hbm,
             sidx_v, didx_v, rows_v, zbuf_v, y_sh, agg_sh, gsem, ssem):
    cid = lax.axis_index("c")
    sid = lax.axis_index("s")
    wid = sid * NC + cid

    zero16 = jnp.zeros((16,), jnp.float32)
    for i in range(32):
        for j in range(W // 8 // 2):
            zbuf_v[i, pl.ds(j * 16, 16)] = zero16
        zbuf_v[i, pl.ds(W - 16, 16)] = zero16

    def start_gather(i, b):
        pltpu.async_copy(y_sh.at[sidx_v.at[i]], rows_v.at[b], gsem.at[b])

    def wait_gather(b):
        pltpu.make_async_copy(y_sh.at[sidx_v.at[0]], rows_v.at[b],
                              gsem.at[b]).wait()

    for p in range(2):
        pltpu.sync_copy(
            y_hbm.at[pl.ds(sid * SRS, SRS), pl.ds(p * W, W)],
            y_sh.at[pl.ds(sid * SRS, SRS)])

        def zloop(k, c):
            pltpu.sync_copy(zbuf_v,
                            agg_sh.at[pl.ds(sid * RPS + k * 32, 32)])
            return c
        lax.fori_loop(0, RPS // 32, zloop, 0)
        plsc.subcore_barrier()

        for h in range(2):
            cbase = wid * CH + h * HCH
            pltpu.sync_copy(src2_hbm.at[pl.ds(cbase, HCH)], sidx_v)
            pltpu.sync_copy(dst2_hbm.at[pl.ds(cbase, HCH)], didx_v)
            start_gather(0, 0)

            def eloop(g, c2):
                for b in range(2):
                    i = g * 2 + b
                    wait_gather(b)
                    pltpu.async_copy(rows_v.at[b],
                                     agg_sh.at[didx_v.at[i]],
                                     ssem.at[b], add=True)
                    start_gather((i + 1) % HCH, 1 - b)
                    pltpu.make_async_copy(rows_v.at[b],
                                          agg_sh.at[didx_v.at[i]],
                                          ssem.at[b]).wait()
                return c2
            lax.fori_loop(0, HCH // 2, eloop, 0)
            wait_gather(0)
        plsc.subcore_barrier()

        pltpu.sync_copy(agg_sh.at[pl.ds(sid * RPS, RPS)],
                        agg_hbm.at[cid].at[p].at[pl.ds(sid * RPS, RPS)])


def _epilogue_body(a_ref, r_ref, out_ref):
    sa = a_ref[0, 0] + a_ref[1, 0]
    sb = a_ref[0, 1] + a_ref[1, 1]
    cnt = jnp.maximum(sb[:, CC:CC + 1], 1.0)
    mean = jnp.concatenate([sa, sb[:, :CC]], axis=1) / cnt
    out = mean + r_ref[...]
    nrm = jnp.sqrt(jnp.sum(out * out, axis=1, keepdims=True))
    out_ref[...] = out / jnp.maximum(nrm, 1e-12)


def kernel(x, edge_index, W1l, b1l, W1r, bn_gamma, bn_beta, W2l, b2l, W2r):
    del W1l, b1l, W1r

    src = edge_index[0]
    dst = edge_index[1]
    pad = E_PAD - E
    srcp = jnp.concatenate([src, jnp.zeros((pad,), jnp.int32)])
    dstp = jnp.concatenate([dst, jnp.full((pad,), N_PAD - 1, jnp.int32)])
    src2 = srcp.reshape(NW * CH, CHUNK)
    dst2 = dstp.reshape(NW * CH, CHUNK)

    ycat, r = pl.pallas_call(
        _prologue_body,
        out_shape=(
            jax.ShapeDtypeStruct((N, DA), jnp.float32),
            jax.ShapeDtypeStruct((N, H), jnp.float32),
        ),
    )(x, bn_gamma, bn_beta, W2l, W2r, b2l)

    mesh = plsc.VectorSubcoreMesh(core_axis_name="c", subcore_axis_name="s")
    agg2 = pl.kernel(
        _sc_body,
        out_type=jax.ShapeDtypeStruct((NC, 2, N_PAD, W), jnp.float32),
        mesh=mesh,
        scratch_types=[
            pltpu.VMEM((HCH, CHUNK), jnp.int32),
            pltpu.VMEM((HCH, CHUNK), jnp.int32),
            pltpu.VMEM((2, CHUNK, W), jnp.float32),
            pltpu.VMEM((32, W), jnp.float32),
            pltpu.VMEM_SHARED((N_PAD, W), jnp.float32),
            pltpu.VMEM_SHARED((N_PAD, W), jnp.float32),
            pltpu.SemaphoreType.DMA((2,)),
            pltpu.SemaphoreType.DMA((2,)),
        ],
        compiler_params=pltpu.CompilerParams(use_tc_tiling_on_sc=False),
    )(ycat, src2, dst2)

    out = pl.pallas_call(
        _epilogue_body,
        grid=(1,),
        in_specs=[
            pl.BlockSpec((NC, 2, N, W), lambda i: (0, 0, 0, 0)),
            pl.BlockSpec((N, H), lambda i: (0, 0)),
        ],
        out_specs=pl.BlockSpec((N, H), lambda i: (0, 0)),
        out_shape=jax.ShapeDtypeStruct((N, H), jnp.float32),
    )(agg2, r)
    return out
